# baseline (speedup 1.0000x reference)
import functools

import jax
import jax.numpy as jnp
from jax import lax
from jax.experimental import pallas as pl
from jax.experimental.pallas import tpu as pltpu
from jax.experimental.pallas import tpu_sc as plsc

VOCAB1 = 1001
D = 16
NCAT = 26
B = 16384

_NC = 2
_NS = 16
_NW = _NC * _NS
_BPW = B // _NW
_INV = 1.0 / (1.0 + 1e-6)


def _sc_body(item_id_hbm, cat_hbm, vn_hbm, cn_hbm, tn_hbm,
             table_hbm, w1_hbm,
             emb_out, add_out,
             idx_v, cat_v, vn_v, cn_v, tn_v, w_v,
             rows_v, rows2_v, table_s, w1_s,
             semi, semc, semv, semn, semt, semw, sem1, sem2, semo,
             sems1, sems2):
    sid = lax.axis_index("s")
    wid = sid * _NC + lax.axis_index("c")
    base = wid * _BPW
    cpi = pltpu.async_copy(item_id_hbm.at[pl.ds(base, _BPW)], idx_v, semi)
    cpc = pltpu.async_copy(cat_hbm.at[pl.ds(base, _BPW)], cat_v, semc)
    cpv = pltpu.async_copy(vn_hbm.at[pl.ds(base, _BPW)], vn_v, semv)
    cpn = pltpu.async_copy(cn_hbm.at[pl.ds(base, _BPW)], cn_v, semn)
    cpt = pltpu.async_copy(tn_hbm.at[pl.ds(base, _BPW)], tn_v, semt)
    cpw = pltpu.async_copy(w1_hbm.at[pl.ds(D + NCAT, 3)], w_v, semw)

    @pl.when(sid == 0)
    def _stage():
        cps1 = pltpu.async_copy(table_hbm, table_s, sems1)
        cps2 = pltpu.async_copy(w1_hbm, w1_s, sems2)
        cps1.wait()
        cps2.wait()

    plsc.subcore_barrier()

    cpi.wait()
    cp1 = pltpu.async_copy(table_s.at[idx_v], rows_v, sem1)

    cpc.wait()

    @plsc.parallel_loop(0, _BPW // 16, unroll=8)
    def shift_body(j):
        cat_v[pl.ds(j * 16, 16)] = cat_v[pl.ds(j * 16, 16)] + D

    cp2 = pltpu.async_copy(w1_s.at[cat_v], rows2_v, sem2)

    cp1.wait()
    cpo = pltpu.async_copy(rows_v, emb_out.at[pl.ds(base, _BPW)], semo)

    cpv.wait()
    cpn.wait()
    cpt.wait()
    cpw.wait()
    cp2.wait()

    inv = jnp.float32(_INV)
    wv0 = w_v[0] * inv
    wv1 = w_v[1] * inv
    wv2 = w_v[2] * inv

    @plsc.parallel_loop(0, _BPW, unroll=16)
    def row_body(i):
        bidx = jnp.full((16,), i, jnp.int32)
        r = rows2_v[i] + wv0 * plsc.load_gather(vn_v, [bidx])
        r = r + wv1 * plsc.load_gather(cn_v, [bidx])
        r = r + wv2 * plsc.load_gather(tn_v, [bidx])
        rows2_v[i] = r

    cpa = pltpu.async_copy(rows2_v, add_out.at[pl.ds(base, _BPW)], semo)
    cpo.wait()
    cpa.wait()


@functools.lru_cache(maxsize=1)
def _get_sc_kernel():
    mesh = plsc.VectorSubcoreMesh(core_axis_name="c", subcore_axis_name="s")
    return pl.kernel(
        _sc_body,
        out_type=[jax.ShapeDtypeStruct((B, D), jnp.float32),
                  jax.ShapeDtypeStruct((B, D), jnp.float32)],
        mesh=mesh,
        scratch_types=[pltpu.VMEM((_BPW,), jnp.int32),
                       pltpu.VMEM((_BPW,), jnp.int32),
                       pltpu.VMEM((_BPW,), jnp.float32),
                       pltpu.VMEM((_BPW,), jnp.float32),
                       pltpu.VMEM((_BPW,), jnp.float32),
                       pltpu.VMEM((3, D), jnp.float32),
                       pltpu.VMEM((_BPW, D), jnp.float32),
                       pltpu.VMEM((_BPW, D), jnp.float32),
                       pltpu.VMEM_SHARED((VOCAB1, D), jnp.float32),
                       pltpu.VMEM_SHARED((D + NCAT + 3, D), jnp.float32)]
                      + [pltpu.SemaphoreType.DMA] * 11,
        compiler_params=pltpu.CompilerParams(use_tc_tiling_on_sc=False,
                                             needs_layout_passes=False),
    )


_BP = B // 8


def _mlp_body(emb_ref, add_ref, w1_ref, b1_ref, w2_ref, b2_ref, out_ref):
    row128 = lax.broadcasted_iota(jnp.int32, (128, D), 0)
    col16 = lax.broadcasted_iota(jnp.int32, (128, D), 1)
    p = (col16 == lax.rem(row128, D)).astype(jnp.float32)
    blk_r = lax.div(lax.broadcasted_iota(jnp.int32, (128, 128), 0), D)
    blk_c = lax.div(lax.broadcasted_iota(jnp.int32, (128, 128), 1), D)
    bd_mask = (blk_r == blk_c)

    w1a_t = jnp.dot(jnp.dot(p, w1_ref[:D, :],
                            preferred_element_type=jnp.float32), p.T,
                    preferred_element_type=jnp.float32)
    w1a_big = jnp.where(bd_mask, w1a_t, jnp.float32(0.0))
    w2_t = jnp.dot(jnp.dot(p, w2_ref[:], preferred_element_type=jnp.float32),
                   p.T, preferred_element_type=jnp.float32)
    w2_big = jnp.where(bd_mask, w2_t, jnp.float32(0.0))
    b1_big = jnp.dot(b1_ref[:].reshape(1, D), p.T,
                     preferred_element_type=jnp.float32)
    b2_big = jnp.dot(b2_ref[:].reshape(1, D), p.T,
                     preferred_element_type=jnp.float32)

    acc = jnp.dot(emb_ref[:], w1a_big, preferred_element_type=jnp.float32)
    acc = acc + add_ref[:] + b1_big
    h = jnp.maximum(acc, jnp.float32(0.0))
    out_ref[:] = (jnp.dot(h, w2_big, preferred_element_type=jnp.float32)
                  + b2_big)


_TBLK = _BP // 4

_mlp_grid_spec = dict(
    grid=(4,),
    in_specs=[
        pl.BlockSpec((_TBLK, 128), lambda i: (i, 0)),
        pl.BlockSpec((_TBLK, 128), lambda i: (i, 0)),
        pl.BlockSpec((D + NCAT + 3, D), lambda i: (0, 0)),
        pl.BlockSpec((D,), lambda i: (0,)),
        pl.BlockSpec((D, D), lambda i: (0, 0)),
        pl.BlockSpec((D,), lambda i: (0,)),
    ],
    out_specs=pl.BlockSpec((_TBLK, 128), lambda i: (i, 0)),
    out_shape=jax.ShapeDtypeStruct((_BP, 128), jnp.float32),
)

_mlp = pl.pallas_call(_mlp_body, **_mlp_grid_spec)


def kernel(item_id, category, view_count, click_count, title_length,
           item_table, W1, b1, W2, b2):
    emb, addrow = _get_sc_kernel()(
        item_id.astype(jnp.int32), category.astype(jnp.int32),
        view_count, click_count, title_length, item_table, W1)
    out_p = _mlp(emb.reshape(_BP, 128), addrow.reshape(_BP, 128), W1, b1,
                 W2, b2)
    return out_p.reshape(B, D)


# speedup vs baseline: 1.0429x; 1.0429x over previous
import functools

import jax
import jax.numpy as jnp
from jax import lax
from jax.experimental import pallas as pl
from jax.experimental.pallas import tpu as pltpu
from jax.experimental.pallas import tpu_sc as plsc

VOCAB1 = 1001
D = 16
NCAT = 26
B = 16384

_NC = 2
_NS = 16
_NW = _NC * _NS
_BPW = B // _NW
_INV = 1.0 / (1.0 + 1e-6)


def _sc_body(item_id_hbm, cat_hbm, vn_hbm, cn_hbm, tn_hbm,
             table_hbm, w1_hbm,
             emb_out, add_out,
             idx_v, cat_v, vn_v, cn_v, tn_v, w_v,
             rows_v, rows2_v, table_s, w1_s,
             semi, semc, semv, semn, semt, semw, sem1, sem2, semo,
             sems1, sems2):
    sid = lax.axis_index("s")
    wid = sid * _NC + lax.axis_index("c")
    base = wid * _BPW
    cpi = pltpu.async_copy(item_id_hbm.at[pl.ds(base, _BPW)], idx_v, semi)
    cpc = pltpu.async_copy(cat_hbm.at[pl.ds(base, _BPW)], cat_v, semc)
    cpv = pltpu.async_copy(vn_hbm.at[pl.ds(base, _BPW)], vn_v, semv)
    cpn = pltpu.async_copy(cn_hbm.at[pl.ds(base, _BPW)], cn_v, semn)
    cpt = pltpu.async_copy(tn_hbm.at[pl.ds(base, _BPW)], tn_v, semt)
    cpw = pltpu.async_copy(w1_hbm.at[pl.ds(D + NCAT, 3)], w_v, semw)

    @pl.when(sid == 0)
    def _stage():
        cps1 = pltpu.async_copy(table_hbm, table_s, sems1)
        cps2 = pltpu.async_copy(w1_hbm, w1_s, sems2)
        cps1.wait()
        cps2.wait()

    plsc.subcore_barrier()

    cpi.wait()
    cp1 = pltpu.async_copy(table_s.at[idx_v], rows_v, sem1)

    cpc.wait()

    @plsc.parallel_loop(0, _BPW // 16, unroll=8)
    def shift_body(j):
        cat_v[pl.ds(j * 16, 16)] = cat_v[pl.ds(j * 16, 16)] + D

    cp2 = pltpu.async_copy(w1_s.at[cat_v], rows2_v, sem2)

    cp1.wait()
    cpo = pltpu.async_copy(rows_v, emb_out.at[pl.ds(base, _BPW)], semo)

    cpv.wait()
    cpn.wait()
    cpt.wait()
    cpw.wait()
    cp2.wait()

    inv = jnp.float32(_INV)
    wv0 = w_v[0] * inv
    wv1 = w_v[1] * inv
    wv2 = w_v[2] * inv

    @plsc.parallel_loop(0, _BPW, unroll=16)
    def row_body(i):
        bidx = jnp.full((16,), i, jnp.int32)
        r = rows2_v[i] + wv0 * plsc.load_gather(vn_v, [bidx])
        r = r + wv1 * plsc.load_gather(cn_v, [bidx])
        r = r + wv2 * plsc.load_gather(tn_v, [bidx])
        rows2_v[i] = r

    cpa = pltpu.async_copy(rows2_v, add_out.at[pl.ds(base, _BPW)], semo)
    cpo.wait()
    cpa.wait()


@functools.lru_cache(maxsize=1)
def _get_sc_kernel():
    mesh = plsc.VectorSubcoreMesh(core_axis_name="c", subcore_axis_name="s")
    return pl.kernel(
        _sc_body,
        out_type=[jax.ShapeDtypeStruct((B, D), jnp.float32),
                  jax.ShapeDtypeStruct((B, D), jnp.float32)],
        mesh=mesh,
        scratch_types=[pltpu.VMEM((_BPW,), jnp.int32),
                       pltpu.VMEM((_BPW,), jnp.int32),
                       pltpu.VMEM((_BPW,), jnp.float32),
                       pltpu.VMEM((_BPW,), jnp.float32),
                       pltpu.VMEM((_BPW,), jnp.float32),
                       pltpu.VMEM((3, D), jnp.float32),
                       pltpu.VMEM((_BPW, D), jnp.float32),
                       pltpu.VMEM((_BPW, D), jnp.float32),
                       pltpu.VMEM_SHARED((VOCAB1, D), jnp.float32),
                       pltpu.VMEM_SHARED((D + NCAT + 3, D), jnp.float32)]
                      + [pltpu.SemaphoreType.DMA] * 11,
        compiler_params=pltpu.CompilerParams(use_tc_tiling_on_sc=False,
                                             needs_layout_passes=False),
    )


_BP = B // 8


def _mlp_body(emb_ref, add_ref, w1_ref, b1_ref, w2_ref, b2_ref, out_ref):
    row128 = lax.broadcasted_iota(jnp.int32, (128, D), 0)
    col16 = lax.broadcasted_iota(jnp.int32, (128, D), 1)
    p = (col16 == lax.rem(row128, D)).astype(jnp.float32)
    blk_r = lax.div(lax.broadcasted_iota(jnp.int32, (128, 128), 0), D)
    blk_c = lax.div(lax.broadcasted_iota(jnp.int32, (128, 128), 1), D)
    bd_mask = (blk_r == blk_c)

    w1a_t = jnp.dot(jnp.dot(p, w1_ref[:D, :],
                            preferred_element_type=jnp.float32), p.T,
                    preferred_element_type=jnp.float32)
    w1a_big = jnp.where(bd_mask, w1a_t, jnp.float32(0.0))
    w2_t = jnp.dot(jnp.dot(p, w2_ref[:], preferred_element_type=jnp.float32),
                   p.T, preferred_element_type=jnp.float32)
    w2_big = jnp.where(bd_mask, w2_t, jnp.float32(0.0))
    b1_big = jnp.dot(b1_ref[:].reshape(1, D), p.T,
                     preferred_element_type=jnp.float32)
    b2_big = jnp.dot(b2_ref[:].reshape(1, D), p.T,
                     preferred_element_type=jnp.float32)

    acc = jnp.dot(emb_ref[:], w1a_big, preferred_element_type=jnp.float32)
    acc = acc + add_ref[:] + b1_big
    h = jnp.maximum(acc, jnp.float32(0.0))
    out_ref[:] = (jnp.dot(h, w2_big, preferred_element_type=jnp.float32)
                  + b2_big)


_mlp_grid_spec = dict(
    grid=(1,),
    in_specs=[
        pl.BlockSpec((_BP, 128), lambda i: (0, 0)),
        pl.BlockSpec((_BP, 128), lambda i: (0, 0)),
        pl.BlockSpec((D + NCAT + 3, D), lambda i: (0, 0)),
        pl.BlockSpec((D,), lambda i: (0,)),
        pl.BlockSpec((D, D), lambda i: (0, 0)),
        pl.BlockSpec((D,), lambda i: (0,)),
    ],
    out_specs=pl.BlockSpec((_BP, 128), lambda i: (0, 0)),
    out_shape=jax.ShapeDtypeStruct((_BP, 128), jnp.float32),
)

_mlp = pl.pallas_call(_mlp_body, **_mlp_grid_spec)


def kernel(item_id, category, view_count, click_count, title_length,
           item_table, W1, b1, W2, b2):
    emb, addrow = _get_sc_kernel()(
        item_id.astype(jnp.int32), category.astype(jnp.int32),
        view_count, click_count, title_length, item_table, W1)
    out_p = _mlp(emb.reshape(_BP, 128), addrow.reshape(_BP, 128), W1, b1,
                 W2, b2)
    return out_p.reshape(B, D)
